# Initial kernel scaffold; baseline (speedup 1.0000x reference)
import jax
import jax.numpy as jnp
from jax.experimental import pallas as pl


def kernel(x, edge_index, W, b):
    raise NotImplementedError("write your pallas kernel here")


# speedup vs baseline: 12.0935x; 12.0935x over previous
import functools

import jax
import jax.numpy as jnp
from jax import lax
from jax.experimental import pallas as pl
from jax.experimental.pallas import tpu as pltpu
from jax.experimental.pallas import tpu_sc as plsc

N = 10000
E = 160000
D_IN = 256
D_H = 256
HALF = 128

NC = 2
NS = 16
L = 16

RPT = 640
RPT_LAST = N - RPT * (NS - 1)
ROWS_PAD = RPT * NS

CHUNK = 128

EPT1 = E // (NC * NS)
NFULL1 = EPT1 // CHUNK
TAIL1 = EPT1 - NFULL1 * CHUNK

EPT3 = E // NS
NFULL3 = EPT3 // CHUNK
TAIL3 = EPT3 - NFULL3 * CHUNK


def _mesh():
    return plsc.VectorSubcoreMesh(core_axis_name="c", subcore_axis_name="s",
                                  num_cores=NC, num_subcores=NS)


def _row_slab(s):
    return s * RPT, s == NS - 1


def _deg_count(src_idx, dst_idx):
    @functools.partial(
        pl.kernel,
        out_type=jax.ShapeDtypeStruct((NC * N,), jnp.float32),
        mesh=_mesh(),
        scratch_types=[
            pltpu.VMEM_SHARED((ROWS_PAD,), jnp.float32),
            pltpu.VMEM((RPT,), jnp.float32),
            pltpu.VMEM((CHUNK,), jnp.float32),
            pltpu.VMEM((CHUNK,), jnp.int32),
            pltpu.VMEM((TAIL1,), jnp.int32),
        ],
    )
    def k(dst_hbm, deg2_hbm, deg, zerov, onesb, dstb, dstt):
        c = lax.axis_index("c")
        s = lax.axis_index("s")
        base_r, is_last = _row_slab(s)

        for g in range(CHUNK // L):
            onesb[pl.ds(g * L, L)] = jnp.full((L,), 1.0, jnp.float32)
        for g in range(RPT // L):
            zerov[pl.ds(g * L, L)] = jnp.zeros((L,), jnp.float32)
        pltpu.sync_copy(zerov, deg.at[pl.ds(base_r, RPT)])
        plsc.subcore_barrier()

        ebase = (c * NS + s) * EPT1

        def deg_chunk(i, carry):
            off = ebase + i * CHUNK
            pltpu.sync_copy(dst_hbm.at[pl.ds(off, CHUNK)], dstb)
            pltpu.sync_copy(onesb, deg.at[dstb], add=True)
            return carry

        lax.fori_loop(0, NFULL1, deg_chunk, 0)
        offt = ebase + NFULL1 * CHUNK
        pltpu.sync_copy(dst_hbm.at[pl.ds(offt, TAIL1)], dstt)
        pltpu.sync_copy(onesb.at[pl.ds(0, TAIL1)], deg.at[dstt], add=True)
        plsc.subcore_barrier()

        pltpu.sync_copy(deg.at[pl.ds(base_r, RPT)], zerov)

        @pl.when(jnp.logical_not(is_last))
        def _():
            pltpu.sync_copy(zerov,
                            deg2_hbm.at[pl.ds(c * N + base_r, RPT)])

        @pl.when(is_last)
        def _():
            pltpu.sync_copy(zerov.at[pl.ds(0, RPT_LAST)],
                            deg2_hbm.at[pl.ds(c * N + base_r, RPT_LAST)])

    return k(dst_idx)


def _mm_body(x_ref, w_ref, deg_ref, o_ref):
    i = pl.program_id(0)
    d = deg_ref[0, i, :] + deg_ref[1, i, :] + 1.0
    dis = lax.rsqrt(d)
    h = jnp.dot(x_ref[...], w_ref[...], preferred_element_type=jnp.float32)
    o_ref[0, :, :] = h * dis[:, None]


def _matmul_scaled(x, W, degr):
    RB = 400
    return pl.pallas_call(
        _mm_body,
        grid=(N // RB, 2),
        in_specs=[
            pl.BlockSpec((RB, D_IN), lambda i, j: (i, 0)),
            pl.BlockSpec((D_IN, HALF), lambda i, j: (0, j)),
            pl.BlockSpec((NC, N // RB, RB), lambda i, j: (0, 0, 0)),
        ],
        out_specs=pl.BlockSpec((1, RB, HALF), lambda i, j: (j, i, 0)),
        out_shape=jax.ShapeDtypeStruct((2, N, HALF), jnp.float32),
    )(x, W, degr)


def _edge_accumulate(hs2, src_idx, dst_idx):
    @functools.partial(
        pl.kernel,
        out_type=jax.ShapeDtypeStruct((NC, N, HALF), jnp.float32),
        mesh=_mesh(),
        scratch_types=[
            pltpu.VMEM_SHARED((ROWS_PAD, HALF), jnp.float32),
            pltpu.VMEM((CHUNK, HALF), jnp.float32),
            pltpu.VMEM((TAIL3, HALF), jnp.float32),
            pltpu.VMEM((CHUNK,), jnp.int32),
            pltpu.VMEM((CHUNK,), jnp.int32),
            pltpu.VMEM((TAIL3,), jnp.int32),
            pltpu.VMEM((TAIL3,), jnp.int32),
            pltpu.SemaphoreType.DMA,
        ],
    )
    def k(hs_hbm, src_hbm, dst_hbm, acc2_hbm,
          acc, rows, rowst, srcb, dstb, srct, dstt, sem):
        c = lax.axis_index("c")
        s = lax.axis_index("s")
        base_r, is_last = _row_slab(s)

        @pl.when(jnp.logical_not(is_last))
        def _():
            pltpu.sync_copy(hs_hbm.at[c, pl.ds(base_r, RPT)],
                            acc.at[pl.ds(base_r, RPT)])

        @pl.when(is_last)
        def _():
            pltpu.sync_copy(hs_hbm.at[c, pl.ds(base_r, RPT_LAST)],
                            acc.at[pl.ds(base_r, RPT_LAST)])

        plsc.subcore_barrier()

        ebase = s * EPT3

        def edge_chunk(i, carry):
            off = ebase + i * CHUNK
            pltpu.sync_copy(src_hbm.at[pl.ds(off, CHUNK)], srcb)
            pltpu.sync_copy(dst_hbm.at[pl.ds(off, CHUNK)], dstb)
            pltpu.async_copy(hs_hbm.at[c].at[srcb], rows, sem).wait()
            pltpu.sync_copy(rows, acc.at[dstb], add=True)
            return carry

        lax.fori_loop(0, NFULL3, edge_chunk, 0)
        offt = ebase + NFULL3 * CHUNK
        pltpu.sync_copy(src_hbm.at[pl.ds(offt, TAIL3)], srct)
        pltpu.sync_copy(dst_hbm.at[pl.ds(offt, TAIL3)], dstt)
        pltpu.async_copy(hs_hbm.at[c].at[srct], rowst, sem).wait()
        pltpu.sync_copy(rowst, acc.at[dstt], add=True)
        plsc.subcore_barrier()

        @pl.when(jnp.logical_not(is_last))
        def _():
            pltpu.sync_copy(acc.at[pl.ds(base_r, RPT)],
                            acc2_hbm.at[c, pl.ds(base_r, RPT)])

        @pl.when(is_last)
        def _():
            pltpu.sync_copy(acc.at[pl.ds(base_r, RPT_LAST)],
                            acc2_hbm.at[c, pl.ds(base_r, RPT_LAST)])

    return k(hs2, src_idx, dst_idx)


def _ro_body(acc_ref, deg_ref, b_ref, o_ref):
    i = pl.program_id(0)
    d = deg_ref[0, i, :] + deg_ref[1, i, :] + 1.0
    dis = lax.rsqrt(d)
    a = jnp.concatenate([acc_ref[0], acc_ref[1]], axis=1)
    o_ref[...] = jnp.maximum(a * dis[:, None] + b_ref[0, :][None, :], 0.0)


def _readout(acc2, degr, b):
    RB = 400
    return pl.pallas_call(
        _ro_body,
        grid=(N // RB,),
        in_specs=[
            pl.BlockSpec((NC, RB, HALF), lambda i: (0, i, 0)),
            pl.BlockSpec((NC, N // RB, RB), lambda i: (0, 0, 0)),
            pl.BlockSpec((1, D_H), lambda i: (0, 0)),
        ],
        out_specs=pl.BlockSpec((RB, D_H), lambda i: (i, 0)),
        out_shape=jax.ShapeDtypeStruct((N, D_H), jnp.float32),
    )(acc2, degr, b.reshape(1, D_H))


def kernel(x, edge_index, W, b):
    src_idx = edge_index[0]
    dst_idx = edge_index[1]
    deg2 = _deg_count(src_idx, dst_idx)
    degr = deg2.reshape(NC, N // 400, 400)
    hs2 = _matmul_scaled(x, W, degr)
    acc2 = _edge_accumulate(hs2, src_idx, dst_idx)
    return _readout(acc2, degr, b)


# speedup vs baseline: 17.7956x; 1.4715x over previous
import functools

import jax
import jax.numpy as jnp
from jax import lax
from jax.experimental import pallas as pl
from jax.experimental.pallas import tpu as pltpu
from jax.experimental.pallas import tpu_sc as plsc

N = 10000
E = 160000
D_IN = 256
D_H = 256
HALF = 128

NC = 2
NS = 16
L = 16

RPT = 640
RPT_LAST = N - RPT * (NS - 1)
ROWS_PAD = RPT * NS

CHUNK = 128

EPT1 = E // (NC * NS)
NFULL1 = EPT1 // CHUNK
TAIL1 = EPT1 - NFULL1 * CHUNK

CH3 = 64
EPT3 = E // NS
NFULL3 = EPT3 // CH3
TAIL3 = EPT3 - NFULL3 * CH3


def _mesh():
    return plsc.VectorSubcoreMesh(core_axis_name="c", subcore_axis_name="s",
                                  num_cores=NC, num_subcores=NS)


def _row_slab(s):
    return s * RPT, s == NS - 1


def _deg_count(src_idx, dst_idx):
    @functools.partial(
        pl.kernel,
        out_type=jax.ShapeDtypeStruct((NC * N,), jnp.float32),
        mesh=_mesh(),
        scratch_types=[
            pltpu.VMEM_SHARED((ROWS_PAD,), jnp.float32),
            pltpu.VMEM((RPT,), jnp.float32),
            pltpu.VMEM((CHUNK,), jnp.float32),
            pltpu.VMEM((CHUNK,), jnp.int32),
            pltpu.VMEM((TAIL1,), jnp.int32),
        ],
    )
    def k(dst_hbm, deg2_hbm, deg, zerov, onesb, dstb, dstt):
        c = lax.axis_index("c")
        s = lax.axis_index("s")
        base_r, is_last = _row_slab(s)

        for g in range(CHUNK // L):
            onesb[pl.ds(g * L, L)] = jnp.full((L,), 1.0, jnp.float32)
        for g in range(RPT // L):
            zerov[pl.ds(g * L, L)] = jnp.zeros((L,), jnp.float32)
        pltpu.sync_copy(zerov, deg.at[pl.ds(base_r, RPT)])
        plsc.subcore_barrier()

        ebase = (c * NS + s) * EPT1

        def deg_chunk(i, carry):
            off = ebase + i * CHUNK
            pltpu.sync_copy(dst_hbm.at[pl.ds(off, CHUNK)], dstb)
            pltpu.sync_copy(onesb, deg.at[dstb], add=True)
            return carry

        lax.fori_loop(0, NFULL1, deg_chunk, 0)
        offt = ebase + NFULL1 * CHUNK
        pltpu.sync_copy(dst_hbm.at[pl.ds(offt, TAIL1)], dstt)
        pltpu.sync_copy(onesb.at[pl.ds(0, TAIL1)], deg.at[dstt], add=True)
        plsc.subcore_barrier()

        pltpu.sync_copy(deg.at[pl.ds(base_r, RPT)], zerov)

        @pl.when(jnp.logical_not(is_last))
        def _():
            pltpu.sync_copy(zerov,
                            deg2_hbm.at[pl.ds(c * N + base_r, RPT)])

        @pl.when(is_last)
        def _():
            pltpu.sync_copy(zerov.at[pl.ds(0, RPT_LAST)],
                            deg2_hbm.at[pl.ds(c * N + base_r, RPT_LAST)])

    return k(dst_idx)


def _mm_body(x_ref, w_ref, deg_ref, o_ref):
    i = pl.program_id(0)
    d = deg_ref[0, i, :] + deg_ref[1, i, :] + 1.0
    dis = lax.rsqrt(d)
    h = jnp.dot(x_ref[...], w_ref[...], preferred_element_type=jnp.float32)
    o_ref[0, :, :] = h * dis[:, None]


def _matmul_scaled(x, W, degr):
    RB = 400
    return pl.pallas_call(
        _mm_body,
        grid=(N // RB, 2),
        in_specs=[
            pl.BlockSpec((RB, D_IN), lambda i, j: (i, 0)),
            pl.BlockSpec((D_IN, HALF), lambda i, j: (0, j)),
            pl.BlockSpec((NC, N // RB, RB), lambda i, j: (0, 0, 0)),
        ],
        out_specs=pl.BlockSpec((1, RB, HALF), lambda i, j: (j, i, 0)),
        out_shape=jax.ShapeDtypeStruct((2, N, HALF), jnp.float32),
    )(x, W, degr)


def _edge_accumulate(hs2, src_idx, dst_idx):
    @functools.partial(
        pl.kernel,
        out_type=jax.ShapeDtypeStruct((NC, N, HALF), jnp.float32),
        mesh=_mesh(),
        scratch_types=[
            pltpu.VMEM_SHARED((N, HALF), jnp.float32),
            pltpu.VMEM((CH3, HALF), jnp.float32),
            pltpu.VMEM((CH3, HALF), jnp.float32),
            pltpu.VMEM((TAIL3, HALF), jnp.float32),
            pltpu.VMEM((EPT3,), jnp.int32),
            pltpu.VMEM((EPT3,), jnp.int32),
            pltpu.VMEM((CH3,), jnp.int32),
            pltpu.VMEM((CH3,), jnp.int32),
            pltpu.VMEM((CH3,), jnp.int32),
            pltpu.VMEM((CH3,), jnp.int32),
            pltpu.VMEM((TAIL3,), jnp.int32),
            pltpu.VMEM((TAIL3,), jnp.int32),
            pltpu.SemaphoreType.DMA,
            pltpu.SemaphoreType.DMA,
        ],
    )
    def k(hs_hbm, src_hbm, dst_hbm, acc2_hbm,
          acc, rows0, rows1, rowst, srcall, dstall,
          srcb0, srcb1, dstb0, dstb1, srct, dstt, sem0, sem1):
        c = lax.axis_index("c")
        s = lax.axis_index("s")
        base_r, is_last = _row_slab(s)
        ebase = s * EPT3

        pltpu.sync_copy(src_hbm.at[pl.ds(ebase, EPT3)], srcall)
        pltpu.sync_copy(dst_hbm.at[pl.ds(ebase, EPT3)], dstall)

        @pl.when(jnp.logical_not(is_last))
        def _():
            pltpu.sync_copy(hs_hbm.at[c, pl.ds(base_r, RPT)],
                            acc.at[pl.ds(base_r, RPT)])

        @pl.when(is_last)
        def _():
            pltpu.sync_copy(hs_hbm.at[c, pl.ds(base_r, RPT_LAST)],
                            acc.at[pl.ds(base_r, RPT_LAST)])

        plsc.subcore_barrier()

        def stage_idx(i, srcb, dstb):
            off = i * CH3
            for g in range(CH3 // L):
                srcb[pl.ds(g * L, L)] = srcall[pl.ds(off + g * L, L)]
                dstb[pl.ds(g * L, L)] = dstall[pl.ds(off + g * L, L)]

        def fire(srcb, rows, sem):
            return pltpu.async_copy(hs_hbm.at[c].at[srcb], rows, sem)

        stage_idx(0, srcb0, dstb0)
        fire(srcb0, rows0, sem0)

        def pair(p, carry):
            i0 = 2 * p
            stage_idx(i0 + 1, srcb1, dstb1)
            fire(srcb1, rows1, sem1)
            pltpu.make_async_copy(hs_hbm.at[c].at[srcb0], rows0, sem0).wait()
            pltpu.sync_copy(rows0, acc.at[dstb0], add=True)

            @pl.when(p < NFULL3 // 2 - 1)
            def _():
                stage_idx(i0 + 2, srcb0, dstb0)
                fire(srcb0, rows0, sem0)

            pltpu.make_async_copy(hs_hbm.at[c].at[srcb1], rows1, sem1).wait()
            pltpu.sync_copy(rows1, acc.at[dstb1], add=True)
            return carry

        lax.fori_loop(0, NFULL3 // 2, pair, 0)

        offt = ebase + NFULL3 * CH3
        pltpu.sync_copy(src_hbm.at[pl.ds(offt, TAIL3)], srct)
        pltpu.sync_copy(dst_hbm.at[pl.ds(offt, TAIL3)], dstt)
        pltpu.async_copy(hs_hbm.at[c].at[srct], rowst, sem0).wait()
        pltpu.sync_copy(rowst, acc.at[dstt], add=True)
        plsc.subcore_barrier()

        @pl.when(jnp.logical_not(is_last))
        def _():
            pltpu.sync_copy(acc.at[pl.ds(base_r, RPT)],
                            acc2_hbm.at[c, pl.ds(base_r, RPT)])

        @pl.when(is_last)
        def _():
            pltpu.sync_copy(acc.at[pl.ds(base_r, RPT_LAST)],
                            acc2_hbm.at[c, pl.ds(base_r, RPT_LAST)])

    return k(hs2, src_idx, dst_idx)


def _ro_body(acc_ref, deg_ref, b_ref, o_ref):
    i = pl.program_id(0)
    d = deg_ref[0, i, :] + deg_ref[1, i, :] + 1.0
    dis = lax.rsqrt(d)
    a = jnp.concatenate([acc_ref[0], acc_ref[1]], axis=1)
    o_ref[...] = jnp.maximum(a * dis[:, None] + b_ref[0, :][None, :], 0.0)


def _readout(acc2, degr, b):
    RB = 400
    return pl.pallas_call(
        _ro_body,
        grid=(N // RB,),
        in_specs=[
            pl.BlockSpec((NC, RB, HALF), lambda i: (0, i, 0)),
            pl.BlockSpec((NC, N // RB, RB), lambda i: (0, 0, 0)),
            pl.BlockSpec((1, D_H), lambda i: (0, 0)),
        ],
        out_specs=pl.BlockSpec((RB, D_H), lambda i: (i, 0)),
        out_shape=jax.ShapeDtypeStruct((N, D_H), jnp.float32),
    )(acc2, degr, b.reshape(1, D_H))


def kernel(x, edge_index, W, b):
    src_idx = edge_index[0]
    dst_idx = edge_index[1]
    deg2 = _deg_count(src_idx, dst_idx)
    degr = deg2.reshape(NC, N // 400, 400)
    hs2 = _matmul_scaled(x, W, degr)
    acc2 = _edge_accumulate(hs2, src_idx, dst_idx)
    return _readout(acc2, degr, b)
